# Initial kernel scaffold; baseline (speedup 1.0000x reference)
import jax
import jax.numpy as jnp
from jax.experimental import pallas as pl


def kernel(query_source, context, cn_gamma, cn_beta, qn_gamma, qn_beta, on_gamma, on_beta, w_kv, w_q, w_out, gamma):
    raise NotImplementedError("write your pallas kernel here")


# speedup vs baseline: 4.0320x; 4.0320x over previous
import jax
import jax.numpy as jnp
from jax import lax
from jax.experimental import pallas as pl
from jax.experimental.pallas import tpu as pltpu

F32 = jnp.float32
_PREC = lax.Precision.HIGHEST


def _dot(a, b, dims):
    return lax.dot_general(a, b, dimension_numbers=(dims, ((), ())),
                           preferred_element_type=F32, precision=_PREC)


def _ln_axis0(x, g, b):
    m = jnp.mean(x, axis=0, keepdims=True)
    v = jnp.mean((x - m) * (x - m), axis=0, keepdims=True)
    return g * (x - m) / (jnp.sqrt(v) + 1e-06) + b


def _prep_body(qs_ref, ctx_ref, cng_ref, cnb_ref, qng_ref, qnb_ref,
               wkv_ref, wq_ref, q_out, k_out, v_out):
    qsn = _ln_axis0(qs_ref[0], qng_ref[...], qnb_ref[...])
    ctxn = _ln_axis0(ctx_ref[0], cng_ref[...], cnb_ref[...])
    q = _dot(wq_ref[...], qsn, (((1,), (0,))))
    kv = _dot(wkv_ref[...], ctxn, (((1,), (0,))))
    k = kv[:256]
    v = kv[256:]

    def l2n(t):
        nb = t.shape[1]
        t3 = t.reshape(8, 32, nb)
        n = jnp.sqrt(jnp.sum(t3 * t3, axis=1, keepdims=True))
        return (t3 / jnp.maximum(n, 1e-12)).reshape(256, nb)

    q_out[0] = l2n(q)
    k_out[0] = l2n(k)
    v_out[0] = v


def _prep(qs3, ctx3, cng, cnb, qng, qnb, w_kv, w_q):
    b, c, n = qs3.shape
    nb = 512
    grid = (b, n // nb)
    out_sd = jax.ShapeDtypeStruct((b, 256, n), F32)
    vec = pl.BlockSpec((c, 1), lambda i, j: (0, 0))
    return pl.pallas_call(
        _prep_body,
        grid=grid,
        in_specs=[
            pl.BlockSpec((1, c, nb), lambda i, j: (i, 0, j)),
            pl.BlockSpec((1, c, nb), lambda i, j: (i, 0, j)),
            vec, vec, vec, vec,
            pl.BlockSpec((512, c), lambda i, j: (0, 0)),
            pl.BlockSpec((256, c), lambda i, j: (0, 0)),
        ],
        out_specs=[
            pl.BlockSpec((1, 256, nb), lambda i, j: (i, 0, j)),
            pl.BlockSpec((1, 256, nb), lambda i, j: (i, 0, j)),
            pl.BlockSpec((1, 256, nb), lambda i, j: (i, 0, j)),
        ],
        out_shape=[out_sd, out_sd, out_sd],
    )(qs3, ctx3, cng, cnb, qng, qnb, w_kv, w_q)


def _assign_onehot(cent, x):
    c2 = jnp.sum(cent * cent, axis=0, keepdims=True)
    c_aug = jnp.concatenate([cent, -0.5 * c2], axis=0)
    ones = jnp.ones((1, x.shape[1]), F32)
    x_aug = jnp.concatenate([x, ones], axis=0)
    score = _dot(c_aug, x_aug, (((0,), (0,))))
    smax = jnp.max(score, axis=0, keepdims=True)
    iota0 = lax.broadcasted_iota(jnp.int32, score.shape, 0)
    idx = jnp.min(jnp.where(score == smax, iota0, score.shape[0]),
                  axis=0, keepdims=True)
    return jnp.where(iota0 == idx, 1.0, 0.0).astype(F32)


def _kmeans_body(q_ref, c0_ref, cent_out, cent, sums, counts):
    i = pl.program_id(0)
    j = pl.program_id(1)

    @pl.when((i == 0) & (j == 0))
    def _init():
        cent[...] = c0_ref[...]

    @pl.when(j == 0)
    def _zero():
        sums[...] = jnp.zeros_like(sums)
        counts[...] = jnp.zeros_like(counts)

    x = q_ref[0]
    onehot = _assign_onehot(cent[...], x)
    sums[...] += _dot(x, onehot, (((1,), (1,))))
    counts[...] += _dot(jnp.ones((1, x.shape[1]), F32), onehot,
                        (((1,), (1,))))

    @pl.when(j == pl.num_programs(1) - 1)
    def _update():
        cnt = counts[...]
        newc = sums[...] / jnp.maximum(cnt, 1.0)
        cent[...] = jnp.where(cnt > 0, newc, cent[...])

    cent_out[...] = cent[...]


def _kmeans(q_bh, c0):
    bh, d, n = q_bh.shape
    return pl.pallas_call(
        _kmeans_body,
        grid=(5, bh),
        in_specs=[
            pl.BlockSpec((1, d, n), lambda i, j: (j, 0, 0)),
            pl.BlockSpec((d, 256), lambda i, j: (0, 0)),
        ],
        out_specs=pl.BlockSpec((d, 256), lambda i, j: (0, 0)),
        out_shape=jax.ShapeDtypeStruct((d, 256), F32),
        scratch_shapes=[
            pltpu.VMEM((d, 256), F32),
            pltpu.VMEM((d, 256), F32),
            pltpu.VMEM((1, 256), F32),
        ],
    )(q_bh, c0)


def _kdist_body(k_ref, cent_ref, kd_out):
    k = k_ref[0]
    onehot = _assign_onehot(cent_ref[...], k)
    centers = _dot(cent_ref[...], onehot, (((1,), (0,))))
    kd_out[0] = jnp.sum(jnp.abs(centers - k), axis=0, keepdims=True)


def _kdist(k_bh, cent):
    bh, d, n = k_bh.shape
    return pl.pallas_call(
        _kdist_body,
        grid=(bh,),
        in_specs=[
            pl.BlockSpec((1, d, n), lambda j: (j, 0, 0)),
            pl.BlockSpec((d, 256), lambda j: (0, 0)),
        ],
        out_specs=pl.BlockSpec((1, 1, n), lambda j: (j, 0, 0)),
        out_shape=jax.ShapeDtypeStruct((bh, 1, n), F32),
    )(k_bh, cent)


def _excl_cumsum(f, slt, nb):
    n = f.shape[1]
    chunks = []
    carry = jnp.zeros((1, 1), F32)
    for c in range(n // nb):
        fc = f[:, c * nb:(c + 1) * nb]
        chunks.append(_dot(fc, slt, (((1,), (0,)))) + carry)
        carry = carry + jnp.sum(fc, keepdims=True)
    return jnp.concatenate(chunks, axis=1)


def _attend_body(kd_ref, q_ref, k_ref, v_ref, o_out):
    n = q_ref.shape[2]
    topk = 256
    kd = kd_ref[0]
    kdi = lax.bitcast_convert_type(kd, jnp.int32)

    t = jnp.int32(0)
    for bit in range(30, -1, -1):
        t_try = t | jnp.int32(1 << bit)
        cnt = jnp.sum((kdi >= t_try).astype(jnp.int32))
        t = jnp.where(cnt >= topk, t_try, t)

    g = kdi > t
    e = kdi == t
    needed = (topk - jnp.sum(g.astype(jnp.int32))).astype(F32)

    iota_nb = 512
    slt = (lax.broadcasted_iota(jnp.int32, (iota_nb, iota_nb), 0)
           < lax.broadcasted_iota(jnp.int32, (iota_nb, iota_nb), 1)).astype(F32)
    pe = _excl_cumsum(e.astype(F32), slt, iota_nb)
    sel = jnp.where(g | (e & (pe < needed)), 1.0, 0.0).astype(F32)
    ps = _excl_cumsum(sel, slt, iota_nb)

    k_dm = k_ref[0]
    v_dm = v_ref[0]
    iota_p = lax.broadcasted_iota(jnp.int32, (topk, iota_nb), 0)
    ps_i = ps.astype(jnp.int32)
    ksel = jnp.zeros((32, topk), F32)
    vsel = jnp.zeros((32, topk), F32)
    for c in range(n // iota_nb):
        sl = slice(c * iota_nb, (c + 1) * iota_nb)
        oh = jnp.where((iota_p == ps_i[:, sl]) & (sel[:, sl] > 0.5),
                       1.0, 0.0).astype(F32)
        ksel += _dot(k_dm[:, sl], oh, (((1,), (1,))))
        vsel += _dot(v_dm[:, sl], oh, (((1,), (1,))))

    sim = _dot(q_ref[0], ksel, (((0,), (0,))))
    m = jnp.max(sim, axis=1, keepdims=True)
    p = jnp.exp(sim - m)
    attn = p / jnp.sum(p, axis=1, keepdims=True)
    o_out[0] = _dot(vsel, attn, (((1,), (1,))))


def _attend(kd3, q_bh, k_bh, v_bh):
    bh, d, n = q_bh.shape
    blk = pl.BlockSpec((1, d, n), lambda j: (j, 0, 0))
    return pl.pallas_call(
        _attend_body,
        grid=(bh,),
        in_specs=[pl.BlockSpec((1, 1, n), lambda j: (j, 0, 0)), blk, blk, blk],
        out_specs=blk,
        out_shape=jax.ShapeDtypeStruct((bh, d, n), F32),
    )(kd3, q_bh, k_bh, v_bh)


def _post_body(a_ref, qs_ref, wout_ref, ong_ref, onb_ref, gam_ref, out_ref):
    o = _dot(wout_ref[...], a_ref[0], (((1,), (0,))))
    o = _ln_axis0(o, ong_ref[...], onb_ref[...])
    out_ref[0] = gam_ref[...] * o + qs_ref[0]


def _post(attn_dm, qs3, w_out, ong, onb, gam):
    b, c, n = qs3.shape
    nb = 512
    vec = pl.BlockSpec((c, 1), lambda i, j: (0, 0))
    return pl.pallas_call(
        _post_body,
        grid=(b, n // nb),
        in_specs=[
            pl.BlockSpec((1, 256, nb), lambda i, j: (i, 0, j)),
            pl.BlockSpec((1, c, nb), lambda i, j: (i, 0, j)),
            pl.BlockSpec((c, 256), lambda i, j: (0, 0)),
            vec, vec,
            pl.BlockSpec((1, 1), lambda i, j: (0, 0)),
        ],
        out_specs=pl.BlockSpec((1, c, nb), lambda i, j: (i, 0, j)),
        out_shape=jax.ShapeDtypeStruct((b, c, n), F32),
    )(attn_dm, qs3, w_out, ong, onb, gam)


def kernel(query_source, context, cn_gamma, cn_beta, qn_gamma, qn_beta,
           on_gamma, on_beta, w_kv, w_q, w_out, gamma):
    b, c, D, H, W = query_source.shape
    n = D * H * W
    qs3 = query_source.reshape(b, c, n)
    ctx3 = context.reshape(b, c, n)
    cng = cn_gamma.reshape(c, 1)
    cnb = cn_beta.reshape(c, 1)
    qng = qn_gamma.reshape(c, 1)
    qnb = qn_beta.reshape(c, 1)
    ong = on_gamma.reshape(c, 1)
    onb = on_beta.reshape(c, 1)
    gam = gamma.reshape(1, 1)

    q_dm, k_dm, v_dm = _prep(qs3, ctx3, cng, cnb, qng, qnb, w_kv, w_q)
    q_bh = q_dm.reshape(b * 8, 32, n)
    k_bh = k_dm.reshape(b * 8, 32, n)
    v_bh = v_dm.reshape(b * 8, 32, n)

    c0 = q_bh[0, :, :256]
    cent = _kmeans(q_bh, c0)
    kd3 = _kdist(k_bh, cent)
    attn = _attend(kd3, q_bh, k_bh, v_bh)
    out = _post(attn.reshape(b, 256, n), qs3, w_out, ong, onb, gam)
    return out.reshape(b, c, D, H, W)


# speedup vs baseline: 5.3379x; 1.3239x over previous
import jax
import jax.numpy as jnp
from jax import lax
from jax.experimental import pallas as pl
from jax.experimental.pallas import tpu as pltpu

F32 = jnp.float32
_PREC = lax.Precision.HIGHEST


def _dot(a, b, dims, prec=_PREC):
    return lax.dot_general(a, b, dimension_numbers=(dims, ((), ())),
                           preferred_element_type=F32, precision=prec)


def _bsplit(a, nterms):
    terms = []
    r = a
    for _ in range(nterms):
        t = r.astype(jnp.bfloat16)
        terms.append(t)
        r = r - t.astype(F32)
    return terms


def _dot_onehot(a, oh, dims):
    return _dot(a, oh, dims)


def _ln_axis0(x, g, b):
    m = jnp.mean(x, axis=0, keepdims=True)
    v = jnp.mean((x - m) * (x - m), axis=0, keepdims=True)
    return g * (x - m) / (jnp.sqrt(v) + 1e-06) + b


def _prep_body(qs_ref, ctx_ref, cng_ref, cnb_ref, qng_ref, qnb_ref,
               wkv_ref, wq_ref, q_out, k_out, v_out):
    qsn = _ln_axis0(qs_ref[0], qng_ref[...], qnb_ref[...])
    ctxn = _ln_axis0(ctx_ref[0], cng_ref[...], cnb_ref[...])
    q = _dot(wq_ref[...], qsn, (((1,), (0,))))
    kv = _dot(wkv_ref[...], ctxn, (((1,), (0,))))
    k = kv[:256]
    v = kv[256:]

    def l2n(t):
        nb = t.shape[1]
        t3 = t.reshape(8, 32, nb)
        n = jnp.sqrt(jnp.sum(t3 * t3, axis=1, keepdims=True))
        return (t3 / jnp.maximum(n, 1e-12)).reshape(256, nb)

    q_out[0] = l2n(q)
    k_out[0] = l2n(k)
    v_out[0] = v


def _prep(qs3, ctx3, cng, cnb, qng, qnb, w_kv, w_q):
    b, c, n = qs3.shape
    nb = 512
    grid = (b, n // nb)
    out_sd = jax.ShapeDtypeStruct((b, 256, n), F32)
    vec = pl.BlockSpec((c, 1), lambda i, j: (0, 0))
    return pl.pallas_call(
        _prep_body,
        grid=grid,
        in_specs=[
            pl.BlockSpec((1, c, nb), lambda i, j: (i, 0, j)),
            pl.BlockSpec((1, c, nb), lambda i, j: (i, 0, j)),
            vec, vec, vec, vec,
            pl.BlockSpec((512, c), lambda i, j: (0, 0)),
            pl.BlockSpec((256, c), lambda i, j: (0, 0)),
        ],
        out_specs=[
            pl.BlockSpec((1, 256, nb), lambda i, j: (i, 0, j)),
            pl.BlockSpec((1, 256, nb), lambda i, j: (i, 0, j)),
            pl.BlockSpec((1, 256, nb), lambda i, j: (i, 0, j)),
        ],
        out_shape=[out_sd, out_sd, out_sd],
    )(qs3, ctx3, cng, cnb, qng, qnb, w_kv, w_q)


def _assign_onehot(cent, x, x_aug=None):
    c2 = jnp.sum(cent * cent, axis=0, keepdims=True)
    c_aug = jnp.concatenate([cent, -0.5 * c2], axis=0)
    if x_aug is None:
        ones = jnp.ones((1, x.shape[1]), F32)
        x_aug = jnp.concatenate([x, ones], axis=0)
    score = _dot(c_aug, x_aug, (((0,), (0,))))
    smax = jnp.max(score, axis=0, keepdims=True)
    iota0 = lax.broadcasted_iota(jnp.int32, score.shape, 0)
    idx = jnp.min(jnp.where(score == smax, iota0, score.shape[0]),
                  axis=0, keepdims=True)
    return jnp.where(iota0 == idx, 1.0, 0.0).astype(F32)


def _kmeans_body(q_ref, c0_ref, cent_out, cent, sums):
    i = pl.program_id(0)
    j = pl.program_id(1)

    @pl.when((i == 0) & (j == 0))
    def _init():
        cent[...] = c0_ref[...]

    @pl.when(j == 0)
    def _zero():
        sums[...] = jnp.zeros_like(sums)

    for s in range(q_ref.shape[0]):
        x = q_ref[s]
        ones = jnp.ones((1, x.shape[1]), F32)
        x_aug = jnp.concatenate([x, ones], axis=0)
        onehot = _assign_onehot(cent[...], x, x_aug)
        sums[...] += _dot_onehot(x_aug, onehot, (((1,), (1,))))

    @pl.when(j == pl.num_programs(1) - 1)
    def _update():
        cnt = sums[32:33]
        newc = sums[:32] / jnp.maximum(cnt, 1.0)
        cent[...] = jnp.where(cnt > 0, newc, cent[...])

    cent_out[...] = cent[...]


def _kmeans(q_bh, c0):
    bh, d, n = q_bh.shape
    slab = 2
    return pl.pallas_call(
        _kmeans_body,
        grid=(5, bh // slab),
        in_specs=[
            pl.BlockSpec((slab, d, n), lambda i, j: (j, 0, 0)),
            pl.BlockSpec((d, 256), lambda i, j: (0, 0)),
        ],
        out_specs=pl.BlockSpec((d, 256), lambda i, j: (0, 0)),
        out_shape=jax.ShapeDtypeStruct((d, 256), F32),
        scratch_shapes=[
            pltpu.VMEM((d, 256), F32),
            pltpu.VMEM((d + 1, 256), F32),
        ],
    )(q_bh, c0)


def _kdist_body(k_ref, cent_ref, kd_out):
    for s in range(k_ref.shape[0]):
        k = k_ref[s]
        onehot = _assign_onehot(cent_ref[...], k)
        centers = _dot_onehot(cent_ref[...], onehot, (((1,), (0,))))
        kd_out[s] = jnp.sum(jnp.abs(centers - k), axis=0, keepdims=True)


def _kdist(k_bh, cent):
    bh, d, n = k_bh.shape
    slab = 2
    return pl.pallas_call(
        _kdist_body,
        grid=(bh // slab,),
        in_specs=[
            pl.BlockSpec((slab, d, n), lambda j: (j, 0, 0)),
            pl.BlockSpec((d, 256), lambda j: (0, 0)),
        ],
        out_specs=pl.BlockSpec((slab, 1, n), lambda j: (j, 0, 0)),
        out_shape=jax.ShapeDtypeStruct((bh, 1, n), F32),
    )(k_bh, cent)


def _excl_cumsum(f, slt, nb):
    r, n = f.shape
    chunks = []
    carry = jnp.zeros((r, 1), F32)
    for c in range(n // nb):
        fc = f[:, c * nb:(c + 1) * nb]
        chunks.append(_dot(fc, slt, (((1,), (0,)))) + carry)
        carry = carry + jnp.sum(fc, axis=1, keepdims=True)
    return jnp.concatenate(chunks, axis=1)


def _select_body(kd_ref, sel_out, ps_out):
    topk = 256
    kd = kd_ref[...]
    bh = kd.shape[0]
    kdi = lax.bitcast_convert_type(kd, jnp.int32)

    t = jnp.zeros((bh, 1), jnp.int32)
    for bit in range(30, -1, -1):
        t_try = t | jnp.int32(1 << bit)
        cnt = jnp.sum((kdi >= t_try).astype(jnp.int32), axis=1, keepdims=True)
        t = jnp.where(cnt >= topk, t_try, t)

    g = kdi > t
    e = kdi == t
    needed = (topk - jnp.sum(g.astype(jnp.int32), axis=1,
                             keepdims=True)).astype(F32)

    nb = 512
    slt = (lax.broadcasted_iota(jnp.int32, (nb, nb), 0)
           < lax.broadcasted_iota(jnp.int32, (nb, nb), 1)).astype(F32)
    pe = _excl_cumsum(e.astype(F32), slt, nb)
    sel = jnp.where(g | (e & (pe < needed)), 1.0, 0.0).astype(F32)
    sel_out[...] = sel
    ps_out[...] = _excl_cumsum(sel, slt, nb)


def _select(kd2):
    bh, n = kd2.shape
    sd = jax.ShapeDtypeStruct((bh, n), F32)
    return pl.pallas_call(
        _select_body,
        in_specs=[pl.BlockSpec((bh, n), lambda: (0, 0))],
        out_specs=[pl.BlockSpec((bh, n), lambda: (0, 0))] * 2,
        out_shape=[sd, sd],
    )(kd2)


def _gather_body(sel_ref, ps_ref, k_ref, v_ref, kv_out):
    topk = 256
    nb = 512
    n = k_ref.shape[2]
    sel = sel_ref[0]
    ps_i = ps_ref[0].astype(jnp.int32)
    kv = jnp.concatenate([k_ref[0], v_ref[0]], axis=0)
    iota_p = lax.broadcasted_iota(jnp.int32, (topk, nb), 0)
    acc = jnp.zeros((64, topk), F32)
    for c in range(n // nb):
        sl = slice(c * nb, (c + 1) * nb)
        oh = jnp.where((iota_p == ps_i[:, sl]) & (sel[:, sl] > 0.5),
                       1.0, 0.0).astype(F32)
        acc += _dot(kv[:, sl], oh, (((1,), (1,))))
    kv_out[0] = acc


def _gather(sel3, ps3, k_bh, v_bh):
    bh, d, n = k_bh.shape
    blk = pl.BlockSpec((1, d, n), lambda j: (j, 0, 0))
    row = pl.BlockSpec((1, 1, n), lambda j: (j, 0, 0))
    return pl.pallas_call(
        _gather_body,
        grid=(bh,),
        in_specs=[row, row, blk, blk],
        out_specs=pl.BlockSpec((1, 2 * d, 256), lambda j: (j, 0, 0)),
        out_shape=jax.ShapeDtypeStruct((bh, 2 * d, 256), F32),
    )(sel3, ps3, k_bh, v_bh)


def _attn_body(q_ref, kv_ref, o_out):
    ksel = kv_ref[0, :32]
    vsel = kv_ref[0, 32:]
    sim = _dot(q_ref[0], ksel, (((0,), (0,))))
    p = jnp.exp(sim)
    s = _dot(jnp.ones((1, 256), F32), p, (((1,), (1,))))
    o = _dot(vsel, p, (((1,), (1,))))
    o_out[0] = o * (1.0 / s)


def _attn(q_bh, kv_sel):
    bh, d, n = q_bh.shape
    nq = 2048
    return pl.pallas_call(
        _attn_body,
        grid=(bh, n // nq),
        in_specs=[
            pl.BlockSpec((1, d, nq), lambda j, c: (j, 0, c)),
            pl.BlockSpec((1, 2 * d, 256), lambda j, c: (j, 0, 0)),
        ],
        out_specs=pl.BlockSpec((1, d, nq), lambda j, c: (j, 0, c)),
        out_shape=jax.ShapeDtypeStruct((bh, d, n), F32),
    )(q_bh, kv_sel)


def _post_body(a_ref, qs_ref, wout_ref, ong_ref, onb_ref, gam_ref, out_ref):
    o = _dot(wout_ref[...], a_ref[0], (((1,), (0,))))
    o = _ln_axis0(o, ong_ref[...], onb_ref[...])
    out_ref[0] = gam_ref[...] * o + qs_ref[0]


def _post(attn_dm, qs3, w_out, ong, onb, gam):
    b, c, n = qs3.shape
    nb = 512
    vec = pl.BlockSpec((c, 1), lambda i, j: (0, 0))
    return pl.pallas_call(
        _post_body,
        grid=(b, n // nb),
        in_specs=[
            pl.BlockSpec((1, 256, nb), lambda i, j: (i, 0, j)),
            pl.BlockSpec((1, c, nb), lambda i, j: (i, 0, j)),
            pl.BlockSpec((c, 256), lambda i, j: (0, 0)),
            vec, vec,
            pl.BlockSpec((1, 1), lambda i, j: (0, 0)),
        ],
        out_specs=pl.BlockSpec((1, c, nb), lambda i, j: (i, 0, j)),
        out_shape=jax.ShapeDtypeStruct((b, c, n), F32),
    )(attn_dm, qs3, w_out, ong, onb, gam)


def kernel(query_source, context, cn_gamma, cn_beta, qn_gamma, qn_beta,
           on_gamma, on_beta, w_kv, w_q, w_out, gamma):
    b, c, D, H, W = query_source.shape
    n = D * H * W
    qs3 = query_source.reshape(b, c, n)
    ctx3 = context.reshape(b, c, n)
    cng = cn_gamma.reshape(c, 1)
    cnb = cn_beta.reshape(c, 1)
    qng = qn_gamma.reshape(c, 1)
    qnb = qn_beta.reshape(c, 1)
    ong = on_gamma.reshape(c, 1)
    onb = on_beta.reshape(c, 1)
    gam = gamma.reshape(1, 1)

    q_dm, k_dm, v_dm = _prep(qs3, ctx3, cng, cnb, qng, qnb, w_kv, w_q)
    q_bh = q_dm.reshape(b * 8, 32, n)
    k_bh = k_dm.reshape(b * 8, 32, n)
    v_bh = v_dm.reshape(b * 8, 32, n)

    c0 = q_bh[0, :, :256]
    cent = _kmeans(q_bh, c0)
    kd3 = _kdist(k_bh, cent)
    sel, ps = _select(kd3.reshape(b * 8, n))
    kv_sel = _gather(sel.reshape(b * 8, 1, n), ps.reshape(b * 8, 1, n),
                     k_bh, v_bh)
    attn = _attn(q_bh, kv_sel)
    out = _post(attn.reshape(b, 256, n), qs3, w_out, ong, onb, gam)
    return out.reshape(b, c, D, H, W)
